# baseline (speedup 1.0000x reference)
import functools

import jax
import jax.numpy as jnp
from jax import lax
from jax.experimental import pallas as pl
from jax.experimental.pallas import tpu as pltpu
from jax.experimental.pallas import tpu_sc as plsc

B = 4096
SEQ = 200
D = 64
N = B * SEQ
NW = 32
PER_W = N // NW
CHUNK = 400
NCHUNK = PER_W // CHUNK
DEPTH = 4
K = DEPTH - 1


def _emb_body(ids_hbm, table_hbm, out_hbm, idx_v, rows_v, gsem, osem):
    wid = lax.axis_index("s") * 2 + lax.axis_index("c")
    base = wid * PER_W
    pltpu.sync_copy(ids_hbm.at[pl.ds(base, PER_W)], idx_v)

    def gather(c, b):
        pltpu.async_copy(
            table_hbm.at[idx_v.at[pl.ds(c * CHUNK, CHUNK)]],
            rows_v.at[b],
            gsem.at[b],
        )

    def wait_gather(b):
        pltpu.make_async_copy(
            out_hbm.at[pl.ds(base, CHUNK)], rows_v.at[b], gsem.at[b]
        ).wait()

    def put(c, b):
        pltpu.async_copy(
            rows_v.at[b], out_hbm.at[pl.ds(base + c * CHUNK, CHUNK)], osem.at[b]
        )

    def wait_put(b):
        pltpu.make_async_copy(
            rows_v.at[b], out_hbm.at[pl.ds(base, CHUNK)], osem.at[b]
        ).wait()

    for j in range(K):
        gather(j, j)

    def body(g, _):
        for db in range(DEPTH):
            i = g * DEPTH + db
            b = db
            bn = (db + K) % DEPTH
            @pl.when(i + K < NCHUNK)
            def _():
                @pl.when(i >= 1)
                def _():
                    wait_put(bn)

                gather(i + K, bn)

            wait_gather(b)
            put(i, b)
        return 0

    lax.fori_loop(0, NCHUNK // DEPTH, body, 0, unroll=False)

    for c in range(NCHUNK - DEPTH, NCHUNK):
        wait_put(c % DEPTH)


@jax.jit
def _embed(ids_flat, table):
    mesh = plsc.VectorSubcoreMesh(core_axis_name="c", subcore_axis_name="s")
    return pl.kernel(
        _emb_body,
        out_type=jax.ShapeDtypeStruct((N, D), jnp.float32),
        mesh=mesh,
        scratch_types=[
            pltpu.VMEM((PER_W,), jnp.int32),
            pltpu.VMEM((DEPTH, CHUNK, D), jnp.float32),
            pltpu.SemaphoreType.DMA((DEPTH,)),
            pltpu.SemaphoreType.DMA((DEPTH,)),
        ],
        compiler_params=pltpu.CompilerParams(use_tc_tiling_on_sc=False),
    )(ids_flat, table)


def kernel(token_ids, embed_weight):
    ids_flat = token_ids.reshape(-1)
    out = _embed(ids_flat, embed_weight)
    return out.reshape(B, SEQ, D)


# speedup vs baseline: 1.0033x; 1.0033x over previous
import jax
import jax.numpy as jnp
from jax import lax
from jax.experimental import pallas as pl
from jax.experimental.pallas import tpu as pltpu
from jax.experimental.pallas import tpu_sc as plsc

B = 4096
SEQ = 200
D = 64
NW = 32
ROWS_W = B // NW
DEPTH = 4
K = DEPTH - 1


def _emb_body(ids_hbm, table_hbm, out_hbm, idx_v, rows_v, gsem, osem):
    wid = lax.axis_index("s") * 2 + lax.axis_index("c")
    base = wid * ROWS_W
    pltpu.sync_copy(ids_hbm.at[pl.ds(base, ROWS_W)], idx_v)

    def gather(r, b):
        pltpu.async_copy(
            table_hbm.at[idx_v.at[r]],
            rows_v.at[b],
            gsem.at[b],
        )

    def wait_gather(b):
        pltpu.make_async_copy(
            out_hbm.at[base], rows_v.at[b], gsem.at[b]
        ).wait()

    def put(r, b):
        pltpu.async_copy(rows_v.at[b], out_hbm.at[base + r], osem.at[b])

    def wait_put(b):
        pltpu.make_async_copy(
            rows_v.at[b], out_hbm.at[base], osem.at[b]
        ).wait()

    for j in range(K):
        gather(j, j)

    def body(g, _):
        for db in range(DEPTH):
            i = g * DEPTH + db
            b = db
            bn = (db + K) % DEPTH
            @pl.when(i + K < ROWS_W)
            def _():
                @pl.when(i >= 1)
                def _():
                    wait_put(bn)

                gather(i + K, bn)

            wait_gather(b)
            put(i, b)
        return 0

    lax.fori_loop(0, ROWS_W // DEPTH, body, 0, unroll=False)

    for c in range(ROWS_W - DEPTH, ROWS_W):
        wait_put(c % DEPTH)


@jax.jit
def kernel(token_ids, embed_weight):
    mesh = plsc.VectorSubcoreMesh(core_axis_name="c", subcore_axis_name="s")
    return pl.kernel(
        _emb_body,
        out_type=jax.ShapeDtypeStruct((B, SEQ, D), jnp.float32),
        mesh=mesh,
        scratch_types=[
            pltpu.VMEM((ROWS_W, SEQ), jnp.int32),
            pltpu.VMEM((DEPTH, SEQ, D), jnp.float32),
            pltpu.SemaphoreType.DMA((DEPTH,)),
            pltpu.SemaphoreType.DMA((DEPTH,)),
        ],
        compiler_params=pltpu.CompilerParams(use_tc_tiling_on_sc=False),
    )(token_ids, embed_weight)
